# Initial kernel scaffold; baseline (speedup 1.0000x reference)
import jax
import jax.numpy as jnp
from jax.experimental import pallas as pl


def kernel(hidden_states, router_indices, routing_weights, gate_up_proj, gate_up_proj_bias, down_proj, down_proj_bias):
    raise NotImplementedError("write your pallas kernel here")


# speedup vs baseline: 3.4208x; 3.4208x over previous
import functools

import jax
import jax.numpy as jnp
from jax.experimental import pallas as pl
from jax.experimental.pallas import tpu as pltpu

NUM_EXPERTS = 8
TOP_K = 2
HIDDEN = 1024
EXPERT_DIM = 1024
ALPHA = 1.702
LIMIT = 7.0


def _dense_body(x_ref, wg_ref, wu_ref, wd_ref, bg_ref, bu_ref, bd_ref,
                comb_ref, out_ref, acc_ref):
    e = pl.program_id(1)
    x = x_ref[...]
    gate = jnp.dot(x, wg_ref[0], preferred_element_type=jnp.float32) + bg_ref[0, 0]
    up = jnp.dot(x, wu_ref[0], preferred_element_type=jnp.float32) + bu_ref[0, 0]
    gate = jnp.minimum(gate, LIMIT)
    up = jnp.clip(up, -LIMIT, LIMIT)
    glu = gate * jax.nn.sigmoid(gate * ALPHA)
    h = ((up + 1.0) * glu).astype(jnp.bfloat16)
    y = jnp.dot(h, wd_ref[0], preferred_element_type=jnp.float32) + bd_ref[0, 0]
    y = y * comb_ref[0, 0][:, None]

    @pl.when(e == 0)
    def _init():
        acc_ref[...] = y

    @pl.when(e > 0)
    def _accum():
        acc_ref[...] += y

    @pl.when(e == NUM_EXPERTS - 1)
    def _flush():
        out_ref[...] = acc_ref[...].astype(out_ref.dtype)


def kernel(hidden_states, router_indices, routing_weights, gate_up_proj,
           gate_up_proj_bias, down_proj, down_proj_bias):
    batch, seq, hidden = hidden_states.shape
    T = batch * seq
    x = hidden_states.reshape(T, hidden)

    counts = jnp.zeros((T, NUM_EXPERTS), jnp.float32)
    ids = router_indices.reshape(T, TOP_K)
    counts = counts.at[jnp.arange(T)[:, None], ids].add(1.0)
    combine = counts * routing_weights
    comb_t = combine.T.reshape(NUM_EXPERTS, 1, T)

    wg = gate_up_proj[:, :, 0::2].astype(jnp.bfloat16)
    wu = gate_up_proj[:, :, 1::2].astype(jnp.bfloat16)
    bg = gate_up_proj_bias[:, 0::2].reshape(NUM_EXPERTS, 1, EXPERT_DIM)
    bu = gate_up_proj_bias[:, 1::2].reshape(NUM_EXPERTS, 1, EXPERT_DIM)
    wd = down_proj.astype(jnp.bfloat16)
    bd = down_proj_bias.reshape(NUM_EXPERTS, 1, HIDDEN)

    BT = 1024
    grid = (T // BT, NUM_EXPERTS)
    out = pl.pallas_call(
        _dense_body,
        grid=grid,
        in_specs=[
            pl.BlockSpec((BT, hidden), lambda t, e: (t, 0)),
            pl.BlockSpec((1, hidden, EXPERT_DIM), lambda t, e: (e, 0, 0)),
            pl.BlockSpec((1, hidden, EXPERT_DIM), lambda t, e: (e, 0, 0)),
            pl.BlockSpec((1, EXPERT_DIM, HIDDEN), lambda t, e: (e, 0, 0)),
            pl.BlockSpec((1, 1, EXPERT_DIM), lambda t, e: (e, 0, 0)),
            pl.BlockSpec((1, 1, EXPERT_DIM), lambda t, e: (e, 0, 0)),
            pl.BlockSpec((1, 1, HIDDEN), lambda t, e: (e, 0, 0)),
            pl.BlockSpec((1, 1, BT), lambda t, e: (e, 0, t)),
        ],
        out_specs=pl.BlockSpec((BT, hidden), lambda t, e: (t, 0)),
        out_shape=jax.ShapeDtypeStruct((T, hidden), hidden_states.dtype),
        scratch_shapes=[pltpu.VMEM((BT, hidden), jnp.float32)],
    )(x, wg, wu, wd, bg, bu, bd, comb_t)

    return out.reshape(batch, seq, hidden)


# speedup vs baseline: 3.5773x; 1.0457x over previous
import jax
import jax.numpy as jnp
from jax.experimental import pallas as pl
from jax.experimental.pallas import tpu as pltpu

NUM_EXPERTS = 8
TOP_K = 2
HIDDEN = 1024
EXPERT_DIM = 1024
ALPHA = 1.702
LIMIT = 7.0


def _dense_body(x_ref, wg_ref, wu_ref, wd_ref, bg_ref, bu_ref, bd_ref,
                ids_ref, rw_ref, out_ref, acc_ref):
    e = pl.program_id(1)
    x = x_ref[...]
    gate = jnp.dot(x, wg_ref[0], preferred_element_type=jnp.float32) + bg_ref[0, 0]
    up = jnp.dot(x, wu_ref[0], preferred_element_type=jnp.float32) + bu_ref[0, 0]
    gate = jnp.minimum(gate, LIMIT)
    up = jnp.clip(up, -LIMIT, LIMIT)
    glu = gate * jax.nn.sigmoid(gate * ALPHA)
    h = ((up + 1.0) * glu).astype(jnp.bfloat16)
    y = jnp.dot(h, wd_ref[0], preferred_element_type=jnp.float32) + bd_ref[0, 0]

    ids = ids_ref[...]
    counts = jnp.sum((ids == e).astype(jnp.float32), axis=1)
    ecol = jax.lax.broadcasted_iota(jnp.int32, (1, NUM_EXPERTS), 1)
    w = jnp.sum(rw_ref[...] * (ecol == e).astype(jnp.float32), axis=1)
    y = y * (counts * w)[:, None]

    @pl.when(e == 0)
    def _init():
        acc_ref[...] = y

    @pl.when(e > 0)
    def _accum():
        acc_ref[...] += y

    @pl.when(e == NUM_EXPERTS - 1)
    def _flush():
        out_ref[...] = acc_ref[...].astype(out_ref.dtype)


def kernel(hidden_states, router_indices, routing_weights, gate_up_proj,
           gate_up_proj_bias, down_proj, down_proj_bias):
    batch, seq, hidden = hidden_states.shape
    T = batch * seq
    x = hidden_states.reshape(T, hidden)

    wgu = gate_up_proj.astype(jnp.bfloat16)
    wg = wgu[:, :, 0::2]
    wu = wgu[:, :, 1::2]
    bg = gate_up_proj_bias[:, 0::2].reshape(NUM_EXPERTS, 1, EXPERT_DIM)
    bu = gate_up_proj_bias[:, 1::2].reshape(NUM_EXPERTS, 1, EXPERT_DIM)
    wd = down_proj.astype(jnp.bfloat16)
    bd = down_proj_bias.reshape(NUM_EXPERTS, 1, HIDDEN)
    ids = router_indices.reshape(T, TOP_K).astype(jnp.int32)

    BT = 1024
    grid = (T // BT, NUM_EXPERTS)
    out = pl.pallas_call(
        _dense_body,
        grid=grid,
        in_specs=[
            pl.BlockSpec((BT, hidden), lambda t, e: (t, 0)),
            pl.BlockSpec((1, hidden, EXPERT_DIM), lambda t, e: (e, 0, 0)),
            pl.BlockSpec((1, hidden, EXPERT_DIM), lambda t, e: (e, 0, 0)),
            pl.BlockSpec((1, EXPERT_DIM, HIDDEN), lambda t, e: (e, 0, 0)),
            pl.BlockSpec((1, 1, EXPERT_DIM), lambda t, e: (e, 0, 0)),
            pl.BlockSpec((1, 1, EXPERT_DIM), lambda t, e: (e, 0, 0)),
            pl.BlockSpec((1, 1, HIDDEN), lambda t, e: (e, 0, 0)),
            pl.BlockSpec((BT, TOP_K), lambda t, e: (t, 0)),
            pl.BlockSpec((BT, NUM_EXPERTS), lambda t, e: (t, 0)),
        ],
        out_specs=pl.BlockSpec((BT, hidden), lambda t, e: (t, 0)),
        out_shape=jax.ShapeDtypeStruct((T, hidden), hidden_states.dtype),
        scratch_shapes=[pltpu.VMEM((BT, hidden), jnp.float32)],
    )(x, wg, wu, wd, bg, bu, bd, ids, routing_weights)

    return out.reshape(batch, seq, hidden)


# speedup vs baseline: 14.9560x; 4.1808x over previous
import jax
import jax.numpy as jnp
from jax.experimental import pallas as pl
from jax.experimental.pallas import tpu as pltpu

NUM_EXPERTS = 8
TOP_K = 2
HIDDEN = 1024
EXPERT_DIM = 1024
ALPHA = 1.702
LIMIT = 7.0


def _dense_body(x_ref, wgu_ref, wd2_ref, bgu_ref, bd_ref, ids_ref, rw_ref,
                out_ref, acc_ref):
    e = pl.program_id(1)
    x = x_ref[...]
    gu = jnp.dot(x, wgu_ref[0], preferred_element_type=jnp.float32)
    gu = gu + bgu_ref[0, 0]
    gate = jnp.minimum(gu, LIMIT)
    glu = gate * jax.nn.sigmoid(gate * ALPHA)
    up1 = jnp.clip(gu, -LIMIT, LIMIT) + 1.0
    up1 = pltpu.roll(up1, 2 * EXPERT_DIM - 1, 1)
    lane = jax.lax.broadcasted_iota(jnp.int32, gu.shape, 1)
    h = jnp.where(lane % 2 == 0, glu * up1, 0.0).astype(jnp.bfloat16)
    y = jnp.dot(h, wd2_ref[0], preferred_element_type=jnp.float32) + bd_ref[0, 0]

    ids = ids_ref[...]
    counts = jnp.sum((ids == e).astype(jnp.float32), axis=1)
    ecol = jax.lax.broadcasted_iota(jnp.int32, (1, NUM_EXPERTS), 1)
    w = jnp.sum(rw_ref[...] * (ecol == e).astype(jnp.float32), axis=1)
    y = y * (counts * w)[:, None]

    @pl.when(e == 0)
    def _init():
        acc_ref[...] = y

    @pl.when(e > 0)
    def _accum():
        acc_ref[...] += y

    @pl.when(e == NUM_EXPERTS - 1)
    def _flush():
        out_ref[...] = acc_ref[...].astype(out_ref.dtype)


def kernel(hidden_states, router_indices, routing_weights, gate_up_proj,
           gate_up_proj_bias, down_proj, down_proj_bias):
    batch, seq, hidden = hidden_states.shape
    T = batch * seq
    x = hidden_states.reshape(T, hidden)

    wgu = gate_up_proj.astype(jnp.bfloat16)
    bgu = gate_up_proj_bias.reshape(NUM_EXPERTS, 1, 2 * EXPERT_DIM)
    wd2 = jnp.repeat(down_proj.astype(jnp.bfloat16), 2, axis=1)
    bd = down_proj_bias.reshape(NUM_EXPERTS, 1, HIDDEN)
    ids = router_indices.reshape(T, TOP_K).astype(jnp.int32)

    BT = 1024
    grid = (T // BT, NUM_EXPERTS)
    out = pl.pallas_call(
        _dense_body,
        grid=grid,
        in_specs=[
            pl.BlockSpec((BT, hidden), lambda t, e: (t, 0)),
            pl.BlockSpec((1, hidden, 2 * EXPERT_DIM), lambda t, e: (e, 0, 0)),
            pl.BlockSpec((1, 2 * EXPERT_DIM, HIDDEN), lambda t, e: (e, 0, 0)),
            pl.BlockSpec((1, 1, 2 * EXPERT_DIM), lambda t, e: (e, 0, 0)),
            pl.BlockSpec((1, 1, HIDDEN), lambda t, e: (e, 0, 0)),
            pl.BlockSpec((BT, TOP_K), lambda t, e: (t, 0)),
            pl.BlockSpec((BT, NUM_EXPERTS), lambda t, e: (t, 0)),
        ],
        out_specs=pl.BlockSpec((BT, hidden), lambda t, e: (t, 0)),
        out_shape=jax.ShapeDtypeStruct((T, hidden), hidden_states.dtype),
        scratch_shapes=[pltpu.VMEM((BT, hidden), jnp.float32)],
    )(x, wgu, wd2, bgu, bd, ids, routing_weights)

    return out.reshape(batch, seq, hidden)
